# baseline (speedup 1.0000x reference)
import functools

import jax
import jax.numpy as jnp
from jax import lax
from jax.experimental import pallas as pl
from jax.experimental.pallas import tpu as pltpu
from jax.experimental.pallas import tpu_sc as plsc

_VOCAB = 100000
_D = 64
_HID = 256
_NCLS = 128
_B = 4096
_L = 200

_NC = 2
_NS = 16
_NW = _NC * _NS
_BPT = _B // _NW
_CHUNKS = ((0, 104), (104, 96))
_NV = _D // 16

_ROWS_PER_IT = 4


def _sc_pool_body(x_hbm, emb_hbm, sums_hbm, idx_v, rows_v, out_v, sem0, sem1):
    wid = lax.axis_index("s") * _NC + lax.axis_index("c")
    base = wid * _BPT
    pltpu.sync_copy(x_hbm.at[pl.ds(base, _BPT)], idx_v)
    sems = (sem0, sem1)

    def fire(bag, buf):
        return [
            pltpu.async_copy(
                emb_hbm.at[idx_v.at[bag, pl.ds(off, ln)]],
                rows_v.at[buf, pl.ds(off, ln)],
                sems[buf],
            )
            for off, ln in _CHUNKS
        ]

    def drain(buf):
        for off, ln in _CHUNKS:
            pltpu.make_async_copy(
                emb_hbm.at[idx_v.at[0, pl.ds(off, ln)]],
                rows_v.at[buf, pl.ds(off, ln)],
                sems[buf],
            ).wait()

    def accumulate(pair, half, buf):
        himask = jnp.int32(-65536)

        def row_body(r, accs):
            r0 = r * _ROWS_PER_IT
            new = list(accs)
            for dr in range(_ROWS_PER_IT):
                for cc in range(_NV // 2):
                    v = plsc.bitcast(
                        rows_v[buf, r0 + dr, pl.ds(cc * 32, 32)], jnp.int32)
                    lo = plsc.bitcast(lax.shift_left(v, 16), jnp.float32)
                    hi = plsc.bitcast(lax.bitwise_and(v, himask), jnp.float32)
                    new[2 * cc] = new[2 * cc] + lo
                    new[2 * cc + 1] = new[2 * cc + 1] + hi
            return tuple(new)

        accs = lax.fori_loop(
            0, _L // _ROWS_PER_IT, row_body,
            tuple(jnp.zeros((16,), jnp.float32) for _ in range(_NV)),
        )
        for cc in range(_NV):
            out_v[pair, pl.ds(half * _D + cc * 16, 16)] = accs[cc]

    fire(0, 0)

    def pair_body(i, carry):
        bag = 2 * i
        fire(jnp.minimum(bag + 1, _BPT - 1), 1)
        drain(0)
        accumulate(i, 0, 0)
        fire(jnp.minimum(bag + 2, _BPT - 1), 0)
        drain(1)
        accumulate(i, 1, 1)
        return carry

    lax.fori_loop(0, _BPT // 2, pair_body, 0)
    drain(0)
    pltpu.sync_copy(out_v, sums_hbm.at[pl.ds(base // 2, _BPT // 2)])


@functools.cache
def _sc_pool():
    return pl.kernel(
        _sc_pool_body,
        out_type=jax.ShapeDtypeStruct((_B // 2, 2 * _D), jnp.float32),
        mesh=plsc.VectorSubcoreMesh(core_axis_name="c", subcore_axis_name="s"),
        scratch_types=[
            pltpu.VMEM((_BPT, _L), jnp.int32),
            pltpu.VMEM((2, _L, 2 * _D), jnp.bfloat16),
            pltpu.VMEM((_BPT // 2, 2 * _D), jnp.float32),
            pltpu.SemaphoreType.DMA,
            pltpu.SemaphoreType.DMA,
        ],
        compiler_params=pltpu.CompilerParams(
            use_tc_tiling_on_sc=False, needs_layout_passes=False),
    )


_BV = 2000


def _cast_body(emb_ref, out_ref):
    out_ref[:, :_D] = emb_ref[...].astype(jnp.bfloat16)


def _cast_pad(emb):
    return pl.pallas_call(
        _cast_body,
        grid=(_VOCAB // _BV,),
        in_specs=[pl.BlockSpec((_BV, _D), lambda i: (i, 0))],
        out_specs=pl.BlockSpec((_BV, 2 * _D), lambda i: (i, 0)),
        out_shape=jax.ShapeDtypeStruct((_VOCAB, 2 * _D), jnp.bfloat16),
    )(emb)


_BM = 512


def _mlp_body(sums_ref, x_ref, w1_ref, b1_ref, w2_ref, b2_ref, out_ref):
    cnt = jnp.sum((x_ref[...] != 0).astype(jnp.float32), axis=1, keepdims=True)
    cnt2 = jnp.maximum(cnt, 1.0).reshape(_BM // 2, 2, 1)
    s = sums_ref[...]
    outs = []
    for half in range(2):
        mean = s[:, half * _D:(half + 1) * _D] / cnt2[:, half, :]
        h = jnp.maximum(
            jnp.dot(mean, w1_ref[...], preferred_element_type=jnp.float32)
            + b1_ref[...],
            0.0,
        )
        outs.append(
            jnp.dot(h, w2_ref[...], preferred_element_type=jnp.float32)
            + b2_ref[...]
        )
    out_ref[...] = jnp.stack(outs, axis=1).reshape(_BM, _NCLS)


def _mlp(sums2, x, W1, b1, W2, b2):
    return pl.pallas_call(
        _mlp_body,
        grid=(_B // _BM,),
        in_specs=[
            pl.BlockSpec((_BM // 2, 2 * _D), lambda i: (i, 0)),
            pl.BlockSpec((_BM, _L), lambda i: (i, 0)),
            pl.BlockSpec((_D, _HID), lambda i: (0, 0)),
            pl.BlockSpec((1, _HID), lambda i: (0, 0)),
            pl.BlockSpec((_HID, _NCLS), lambda i: (0, 0)),
            pl.BlockSpec((1, _NCLS), lambda i: (0, 0)),
        ],
        out_specs=pl.BlockSpec((_BM, _NCLS), lambda i: (i, 0)),
        out_shape=jax.ShapeDtypeStruct((_B, _NCLS), jnp.float32),
    )(sums2, x, W1, b1, W2, b2)


_PERM = (
    tuple(range(0, 32, 2)) + tuple(range(1, 32, 2))
    + tuple(range(32, 64, 2)) + tuple(range(33, 64, 2))
)


def kernel(x, lengths, emb, W1, b1, W2, b2):
    del lengths
    x = x.astype(jnp.int32)
    sums2 = _sc_pool()(x, _cast_pad(emb))
    W1p = W1[jnp.array(_PERM, dtype=jnp.int32), :]
    return _mlp(sums2, x, W1p, b1.reshape(1, _HID), W2, b2.reshape(1, _NCLS))


# speedup vs baseline: 1.2913x; 1.2913x over previous
import functools

import jax
import jax.numpy as jnp
from jax import lax
from jax.experimental import pallas as pl
from jax.experimental.pallas import tpu as pltpu
from jax.experimental.pallas import tpu_sc as plsc

_VOCAB = 100000
_D = 64
_HID = 256
_NCLS = 128
_B = 4096
_L = 200

_NC = 2
_NS = 16
_NW = _NC * _NS
_BPT = _B // _NW
_CHUNKS = ((0, 104), (104, 96))
_NV = _D // 16

_ROWS_PER_IT = 4


def _sc_pool_body(x_hbm, emb_hbm, sums_hbm, idx_v, rows_v, out_v, sem0, sem1):
    wid = lax.axis_index("s") * _NC + lax.axis_index("c")
    base = wid * _BPT
    pltpu.sync_copy(x_hbm.at[pl.ds(base, _BPT)], idx_v)
    sems = (sem0, sem1)

    def fire(bag, buf):
        return [
            pltpu.async_copy(
                emb_hbm.at[idx_v.at[bag, pl.ds(off, ln)]],
                rows_v.at[buf, pl.ds(off, ln)],
                sems[buf],
            )
            for off, ln in _CHUNKS
        ]

    def drain(buf):
        for off, ln in _CHUNKS:
            pltpu.make_async_copy(
                emb_hbm.at[idx_v.at[0, pl.ds(off, ln)]],
                rows_v.at[buf, pl.ds(off, ln)],
                sems[buf],
            ).wait()

    def accumulate(pair, half, buf):
        himask = jnp.int32(-65536)

        def row_body(r, accs):
            r0 = r * _ROWS_PER_IT
            new = list(accs)
            for dr in range(_ROWS_PER_IT):
                for cc in range(_NV // 2):
                    v = plsc.bitcast(
                        rows_v[buf, r0 + dr, pl.ds(cc * 32, 32)], jnp.int32)
                    lo = plsc.bitcast(lax.shift_left(v, 16), jnp.float32)
                    hi = plsc.bitcast(lax.bitwise_and(v, himask), jnp.float32)
                    new[2 * cc] = new[2 * cc] + lo
                    new[2 * cc + 1] = new[2 * cc + 1] + hi
            return tuple(new)

        accs = lax.fori_loop(
            0, _L // _ROWS_PER_IT, row_body,
            tuple(jnp.zeros((16,), jnp.float32) for _ in range(_NV)),
        )
        for cc in range(_NV):
            out_v[pair, pl.ds(half * _D + cc * 16, 16)] = accs[cc]

    fire(0, 0)

    def pair_body(i, carry):
        bag = 2 * i
        fire(jnp.minimum(bag + 1, _BPT - 1), 1)
        drain(0)
        accumulate(i, 0, 0)
        fire(jnp.minimum(bag + 2, _BPT - 1), 0)
        drain(1)
        accumulate(i, 1, 1)
        return carry

    lax.fori_loop(0, _BPT // 2, pair_body, 0)
    drain(0)
    pltpu.sync_copy(out_v, sums_hbm.at[pl.ds(base // 2, _BPT // 2)])


@functools.cache
def _sc_pool():
    return pl.kernel(
        _sc_pool_body,
        out_type=jax.ShapeDtypeStruct((_B // 2, 2 * _D), jnp.float32),
        mesh=plsc.VectorSubcoreMesh(core_axis_name="c", subcore_axis_name="s"),
        scratch_types=[
            pltpu.VMEM((_BPT, _L), jnp.int32),
            pltpu.VMEM((2, _L, _D), jnp.bfloat16),
            pltpu.VMEM((_BPT // 2, 2 * _D), jnp.float32),
            pltpu.SemaphoreType.DMA,
            pltpu.SemaphoreType.DMA,
        ],
        compiler_params=pltpu.CompilerParams(
            use_tc_tiling_on_sc=False, needs_layout_passes=False),
    )


_BV = 4000


def _cast_body(emb_ref, out_ref):
    out_ref[...] = emb_ref[...].astype(jnp.bfloat16)


def _cast_tc(emb):
    return pl.pallas_call(
        _cast_body,
        grid=(_VOCAB // _BV,),
        in_specs=[pl.BlockSpec((_BV, _D), lambda i: (i, 0))],
        out_specs=pl.BlockSpec((_BV, _D), lambda i: (i, 0)),
        out_shape=jax.ShapeDtypeStruct((_VOCAB, _D), jnp.bfloat16),
    )(emb)


_BM = 512


def _mlp_body(sums_ref, x_ref, w1_ref, b1_ref, w2_ref, b2_ref, out_ref):
    cnt = jnp.sum((x_ref[...] != 0).astype(jnp.float32), axis=1, keepdims=True)
    cnt2 = jnp.maximum(cnt, 1.0).reshape(_BM // 2, 2, 1)
    s = sums_ref[...]
    outs = []
    for half in range(2):
        mean = s[:, half * _D:(half + 1) * _D] / cnt2[:, half, :]
        h = jnp.maximum(
            jnp.dot(mean, w1_ref[...], preferred_element_type=jnp.float32)
            + b1_ref[...],
            0.0,
        )
        outs.append(
            jnp.dot(h, w2_ref[...], preferred_element_type=jnp.float32)
            + b2_ref[...]
        )
    out_ref[...] = jnp.stack(outs, axis=1).reshape(_BM, _NCLS)


def _mlp(sums2, x, W1, b1, W2, b2):
    return pl.pallas_call(
        _mlp_body,
        grid=(_B // _BM,),
        in_specs=[
            pl.BlockSpec((_BM // 2, 2 * _D), lambda i: (i, 0)),
            pl.BlockSpec((_BM, _L), lambda i: (i, 0)),
            pl.BlockSpec((_D, _HID), lambda i: (0, 0)),
            pl.BlockSpec((1, _HID), lambda i: (0, 0)),
            pl.BlockSpec((_HID, _NCLS), lambda i: (0, 0)),
            pl.BlockSpec((1, _NCLS), lambda i: (0, 0)),
        ],
        out_specs=pl.BlockSpec((_BM, _NCLS), lambda i: (i, 0)),
        out_shape=jax.ShapeDtypeStruct((_B, _NCLS), jnp.float32),
    )(sums2, x, W1, b1, W2, b2)


_PERM = (
    tuple(range(0, 32, 2)) + tuple(range(1, 32, 2))
    + tuple(range(32, 64, 2)) + tuple(range(33, 64, 2))
)


def kernel(x, lengths, emb, W1, b1, W2, b2):
    del lengths
    x = x.astype(jnp.int32)
    sums2 = _sc_pool()(x, _cast_tc(emb))
    W1p = W1[jnp.array(_PERM, dtype=jnp.int32), :]
    return _mlp(sums2, x, W1p, b1.reshape(1, _HID), W2, b2.reshape(1, _NCLS))


# speedup vs baseline: 1.5475x; 1.1983x over previous
import functools

import jax
import jax.numpy as jnp
from jax import lax
from jax.experimental import pallas as pl
from jax.experimental.pallas import tpu as pltpu
from jax.experimental.pallas import tpu_sc as plsc

_VOCAB = 100000
_D = 64
_HID = 256
_NCLS = 128
_B = 4096
_L = 200

_NC = 2
_NS = 16
_NW = _NC * _NS
_BPT = _B // _NW
_CHUNKS = ((0, 104), (104, 96))
_NV = _D // 16

_ROWS_PER_IT = 4


def _sc_pool_body(x_hbm, emb_hbm, sums_hbm, idx_v, rows_v, out_v, sem0, sem1):
    wid = lax.axis_index("s") * _NC + lax.axis_index("c")
    base = wid * _BPT
    pltpu.sync_copy(x_hbm.at[pl.ds(base, _BPT)], idx_v)
    sems = (sem0, sem1)

    def fire(bag, buf):
        return [
            pltpu.async_copy(
                emb_hbm.at[idx_v.at[bag, pl.ds(off, ln)]],
                rows_v.at[buf, pl.ds(off, ln)],
                sems[buf],
            )
            for off, ln in _CHUNKS
        ]

    def drain(buf):
        for off, ln in _CHUNKS:
            pltpu.make_async_copy(
                emb_hbm.at[idx_v.at[0, pl.ds(off, ln)]],
                rows_v.at[buf, pl.ds(off, ln)],
                sems[buf],
            ).wait()

    def accumulate(pair, half, buf):
        def row_body(r, accs):
            r0 = r * _ROWS_PER_IT
            new = list(accs)
            for dr in range(_ROWS_PER_IT):
                for cc in range(_NV):
                    new[cc] = new[cc] + rows_v[buf, r0 + dr, pl.ds(cc * 16, 16)]
            return tuple(new)

        accs = lax.fori_loop(
            0, _L // _ROWS_PER_IT, row_body,
            tuple(jnp.zeros((16,), jnp.float32) for _ in range(_NV)),
        )
        for cc in range(_NV):
            out_v[pair, pl.ds(half * _D + cc * 16, 16)] = accs[cc]

    fire(0, 0)

    def pair_body(i, carry):
        bag = 2 * i
        fire(jnp.minimum(bag + 1, _BPT - 1), 1)
        drain(0)
        accumulate(i, 0, 0)
        fire(jnp.minimum(bag + 2, _BPT - 1), 0)
        drain(1)
        accumulate(i, 1, 1)
        return carry

    lax.fori_loop(0, _BPT // 2, pair_body, 0)
    drain(0)
    pltpu.sync_copy(out_v, sums_hbm.at[pl.ds(base // 2, _BPT // 2)])


@functools.cache
def _sc_pool():
    return pl.kernel(
        _sc_pool_body,
        out_type=jax.ShapeDtypeStruct((_B // 2, 2 * _D), jnp.float32),
        mesh=plsc.VectorSubcoreMesh(core_axis_name="c", subcore_axis_name="s"),
        scratch_types=[
            pltpu.VMEM((_BPT, _L), jnp.int32),
            pltpu.VMEM((2, _L, _D), jnp.float32),
            pltpu.VMEM((_BPT // 2, 2 * _D), jnp.float32),
            pltpu.SemaphoreType.DMA,
            pltpu.SemaphoreType.DMA,
        ],
        compiler_params=pltpu.CompilerParams(
            use_tc_tiling_on_sc=False, needs_layout_passes=False),
    )


_BM = 512


def _mlp_body(sums_ref, x_ref, w1_ref, b1_ref, w2_ref, b2_ref, out_ref):
    cnt = jnp.sum((x_ref[...] != 0).astype(jnp.float32), axis=1, keepdims=True)
    cnt2 = jnp.maximum(cnt, 1.0).reshape(_BM // 2, 2, 1)
    s = sums_ref[...]
    outs = []
    for half in range(2):
        mean = s[:, half * _D:(half + 1) * _D] / cnt2[:, half, :]
        h = jnp.maximum(
            jnp.dot(mean, w1_ref[...], preferred_element_type=jnp.float32)
            + b1_ref[...],
            0.0,
        )
        outs.append(
            jnp.dot(h, w2_ref[...], preferred_element_type=jnp.float32)
            + b2_ref[...]
        )
    out_ref[...] = jnp.stack(outs, axis=1).reshape(_BM, _NCLS)


def _mlp(sums2, x, W1, b1, W2, b2):
    return pl.pallas_call(
        _mlp_body,
        grid=(_B // _BM,),
        in_specs=[
            pl.BlockSpec((_BM // 2, 2 * _D), lambda i: (i, 0)),
            pl.BlockSpec((_BM, _L), lambda i: (i, 0)),
            pl.BlockSpec((_D, _HID), lambda i: (0, 0)),
            pl.BlockSpec((1, _HID), lambda i: (0, 0)),
            pl.BlockSpec((_HID, _NCLS), lambda i: (0, 0)),
            pl.BlockSpec((1, _NCLS), lambda i: (0, 0)),
        ],
        out_specs=pl.BlockSpec((_BM, _NCLS), lambda i: (i, 0)),
        out_shape=jax.ShapeDtypeStruct((_B, _NCLS), jnp.float32),
    )(sums2, x, W1, b1, W2, b2)


def kernel(x, lengths, emb, W1, b1, W2, b2):
    del lengths
    x = x.astype(jnp.int32)
    sums2 = _sc_pool()(x, emb)
    return _mlp(sums2, x, W1, b1.reshape(1, _HID), W2, b2.reshape(1, _NCLS))


# speedup vs baseline: 1.5608x; 1.0086x over previous
import functools

import jax
import jax.numpy as jnp
from jax import lax
from jax.experimental import pallas as pl
from jax.experimental.pallas import tpu as pltpu
from jax.experimental.pallas import tpu_sc as plsc

_VOCAB = 100000
_D = 64
_HID = 256
_NCLS = 128
_B = 4096
_L = 200

_NC = 2
_NS = 16
_NW = _NC * _NS
_BPT = _B // _NW
_CHUNKS = ((0, 104), (104, 96))
_NV = _D // 16

_ROWS_PER_IT = 4


def _sc_pool_body(x_hbm, emb_hbm, sums_hbm, idx_v, rows_v, out_v, sem0, sem1):
    wid = lax.axis_index("s") * _NC + lax.axis_index("c")
    base = wid * _BPT
    pltpu.sync_copy(x_hbm.at[pl.ds(base, _BPT)], idx_v)
    sems = (sem0, sem1)

    def fire(bag, buf):
        return [
            pltpu.async_copy(
                emb_hbm.at[idx_v.at[bag, pl.ds(off, ln)]],
                rows_v.at[buf, pl.ds(off, ln)],
                sems[buf],
            )
            for off, ln in _CHUNKS
        ]

    def drain(buf):
        for off, ln in _CHUNKS:
            pltpu.make_async_copy(
                emb_hbm.at[idx_v.at[0, pl.ds(off, ln)]],
                rows_v.at[buf, pl.ds(off, ln)],
                sems[buf],
            ).wait()

    def accumulate(pair, half, buf):
        def row_body(r, accs):
            r0 = r * _ROWS_PER_IT
            new = list(accs)
            for dr in range(_ROWS_PER_IT):
                for cc in range(_NV):
                    new[cc] = new[cc] + rows_v[buf, r0 + dr, pl.ds(cc * 16, 16)]
            return tuple(new)

        accs = lax.fori_loop(
            0, _L // _ROWS_PER_IT, row_body,
            tuple(jnp.zeros((16,), jnp.float32) for _ in range(_NV)),
        )
        for cc in range(_NV):
            out_v[pair, pl.ds(half * _D + cc * 16, 16)] = accs[cc]

    fire(0, 0)

    def pair_body(i, carry):
        bag = 2 * i
        fire(jnp.minimum(bag + 1, _BPT - 1), 1)
        drain(0)
        accumulate(i, 0, 0)
        fire(jnp.minimum(bag + 2, _BPT - 1), 0)
        drain(1)
        accumulate(i, 1, 1)
        return carry

    lax.fori_loop(0, _BPT // 2, pair_body, 0)
    drain(0)
    pltpu.sync_copy(out_v, sums_hbm.at[pl.ds(base // 2, _BPT // 2)])


@functools.cache
def _sc_pool():
    return pl.kernel(
        _sc_pool_body,
        out_type=jax.ShapeDtypeStruct((_B // 2, 2 * _D), jnp.float32),
        mesh=plsc.VectorSubcoreMesh(core_axis_name="c", subcore_axis_name="s"),
        scratch_types=[
            pltpu.VMEM((_BPT, _L), jnp.int32),
            pltpu.VMEM((2, _L, _D), jnp.float32),
            pltpu.VMEM((_BPT // 2, 2 * _D), jnp.float32),
            pltpu.SemaphoreType.DMA,
            pltpu.SemaphoreType.DMA,
        ],
        compiler_params=pltpu.CompilerParams(
            use_tc_tiling_on_sc=False, needs_layout_passes=False),
    )


_BM = 1024


def _mlp_body(sums_ref, x_ref, w1_ref, b1_ref, w2_ref, b2_ref, out_ref):
    cnt = jnp.sum((x_ref[...] != 0).astype(jnp.float32), axis=1, keepdims=True)
    cnt2 = jnp.maximum(cnt, 1.0).reshape(_BM // 2, 2, 1)
    s = sums_ref[...]
    outs = []
    for half in range(2):
        mean = s[:, half * _D:(half + 1) * _D] / cnt2[:, half, :]
        h = jnp.maximum(
            jnp.dot(mean, w1_ref[...], preferred_element_type=jnp.float32)
            + b1_ref[...],
            0.0,
        )
        outs.append(
            jnp.dot(h, w2_ref[...], preferred_element_type=jnp.float32)
            + b2_ref[...]
        )
    out_ref[...] = jnp.stack(outs, axis=1).reshape(_BM, _NCLS)


def _mlp(sums2, x, W1, b1, W2, b2):
    return pl.pallas_call(
        _mlp_body,
        grid=(_B // _BM,),
        in_specs=[
            pl.BlockSpec((_BM // 2, 2 * _D), lambda i: (i, 0)),
            pl.BlockSpec((_BM, _L), lambda i: (i, 0)),
            pl.BlockSpec((_D, _HID), lambda i: (0, 0)),
            pl.BlockSpec((1, _HID), lambda i: (0, 0)),
            pl.BlockSpec((_HID, _NCLS), lambda i: (0, 0)),
            pl.BlockSpec((1, _NCLS), lambda i: (0, 0)),
        ],
        out_specs=pl.BlockSpec((_BM, _NCLS), lambda i: (i, 0)),
        out_shape=jax.ShapeDtypeStruct((_B, _NCLS), jnp.float32),
    )(sums2, x, W1, b1, W2, b2)


def kernel(x, lengths, emb, W1, b1, W2, b2):
    del lengths
    x = x.astype(jnp.int32)
    sums2 = _sc_pool()(x, emb)
    return _mlp(sums2, x, W1, b1.reshape(1, _HID), W2, b2.reshape(1, _NCLS))


# speedup vs baseline: 1.5654x; 1.0029x over previous
import functools

import jax
import jax.numpy as jnp
from jax import lax
from jax.experimental import pallas as pl
from jax.experimental.pallas import tpu as pltpu
from jax.experimental.pallas import tpu_sc as plsc

_VOCAB = 100000
_D = 64
_HID = 256
_NCLS = 128
_B = 4096
_L = 200

_NC = 2
_NS = 16
_NW = _NC * _NS
_BPT = _B // _NW
_CHUNKS = ((0, 104), (104, 96))
_NV = _D // 16

_ROWS_PER_IT = 4


def _sc_pool_body(x_hbm, emb_hbm, sums_hbm, idx_v, rows_v, out_v, sem0, sem1):
    wid = lax.axis_index("s") * _NC + lax.axis_index("c")
    base = wid * _BPT
    pltpu.sync_copy(x_hbm.at[pl.ds(base, _BPT)], idx_v)
    sems = (sem0, sem1)

    def fire(bag, buf):
        return [
            pltpu.async_copy(
                emb_hbm.at[idx_v.at[bag, pl.ds(off, ln)]],
                rows_v.at[buf, pl.ds(off, ln)],
                sems[buf],
            )
            for off, ln in _CHUNKS
        ]

    def drain(buf):
        for off, ln in _CHUNKS:
            pltpu.make_async_copy(
                emb_hbm.at[idx_v.at[0, pl.ds(off, ln)]],
                rows_v.at[buf, pl.ds(off, ln)],
                sems[buf],
            ).wait()

    def accumulate(pair, half, buf):
        def row_body(r, accs):
            r0 = r * _ROWS_PER_IT
            new = list(accs)
            for dr in range(_ROWS_PER_IT):
                for cc in range(_NV):
                    new[cc] = new[cc] + rows_v[buf, r0 + dr, pl.ds(cc * 16, 16)]
            return tuple(new)

        accs = lax.fori_loop(
            0, _L // _ROWS_PER_IT, row_body,
            tuple(jnp.zeros((16,), jnp.float32) for _ in range(_NV)),
        )
        for cc in range(_NV):
            out_v[pair, pl.ds(half * _D + cc * 16, 16)] = accs[cc]

    fire(0, 0)

    def pair_body(i, carry):
        bag = 2 * i
        fire(jnp.minimum(bag + 1, _BPT - 1), 1)
        drain(0)
        accumulate(i, 0, 0)
        fire(jnp.minimum(bag + 2, _BPT - 1), 0)
        drain(1)
        accumulate(i, 1, 1)
        return carry

    lax.fori_loop(0, _BPT // 2, pair_body, 0)
    drain(0)
    pltpu.sync_copy(out_v, sums_hbm.at[pl.ds(base // 2, _BPT // 2)])


@functools.cache
def _sc_pool():
    return pl.kernel(
        _sc_pool_body,
        out_type=jax.ShapeDtypeStruct((_B // 2, 2 * _D), jnp.float32),
        mesh=plsc.VectorSubcoreMesh(core_axis_name="c", subcore_axis_name="s"),
        scratch_types=[
            pltpu.VMEM((_BPT, _L), jnp.int32),
            pltpu.VMEM((2, _L, _D), jnp.float32),
            pltpu.VMEM((_BPT // 2, 2 * _D), jnp.float32),
            pltpu.SemaphoreType.DMA,
            pltpu.SemaphoreType.DMA,
        ],
        compiler_params=pltpu.CompilerParams(
            use_tc_tiling_on_sc=False, needs_layout_passes=False),
    )


_BM = 2048


def _mlp_body(sums_ref, x_ref, w1_ref, b1_ref, w2_ref, b2_ref, out_ref):
    cnt = jnp.sum((x_ref[...] != 0).astype(jnp.float32), axis=1, keepdims=True)
    cnt2 = jnp.maximum(cnt, 1.0).reshape(_BM // 2, 2, 1)
    s = sums_ref[...]
    outs = []
    for half in range(2):
        mean = s[:, half * _D:(half + 1) * _D] / cnt2[:, half, :]
        h = jnp.maximum(
            jnp.dot(mean, w1_ref[...], preferred_element_type=jnp.float32)
            + b1_ref[...],
            0.0,
        )
        outs.append(
            jnp.dot(h, w2_ref[...], preferred_element_type=jnp.float32)
            + b2_ref[...]
        )
    out_ref[...] = jnp.stack(outs, axis=1).reshape(_BM, _NCLS)


def _mlp(sums2, x, W1, b1, W2, b2):
    return pl.pallas_call(
        _mlp_body,
        grid=(_B // _BM,),
        in_specs=[
            pl.BlockSpec((_BM // 2, 2 * _D), lambda i: (i, 0)),
            pl.BlockSpec((_BM, _L), lambda i: (i, 0)),
            pl.BlockSpec((_D, _HID), lambda i: (0, 0)),
            pl.BlockSpec((1, _HID), lambda i: (0, 0)),
            pl.BlockSpec((_HID, _NCLS), lambda i: (0, 0)),
            pl.BlockSpec((1, _NCLS), lambda i: (0, 0)),
        ],
        out_specs=pl.BlockSpec((_BM, _NCLS), lambda i: (i, 0)),
        out_shape=jax.ShapeDtypeStruct((_B, _NCLS), jnp.float32),
    )(sums2, x, W1, b1, W2, b2)


def kernel(x, lengths, emb, W1, b1, W2, b2):
    del lengths
    x = x.astype(jnp.int32)
    sums2 = _sc_pool()(x, emb)
    return _mlp(sums2, x, W1, b1.reshape(1, _HID), W2, b2.reshape(1, _NCLS))
